# initial kernel scaffold (unmeasured)
import jax
import jax.numpy as jnp
from jax import lax
from jax.experimental import pallas as pl
from jax.experimental.pallas import tpu as pltpu

N_DEV = 4
EPS = 1e-5


def kernel(x, Wp):
    B, Hs, W, C = x.shape
    Cout = Wp.shape[1]
    BH = 32
    n_h = Hs // BH
    inv_n = 1.0 / float(N_DEV * Hs * W)

    def stats_body(x_ref, out_ref):
        h = pl.program_id(0)
        xb = x_ref[...]
        s = jnp.sum(xb, axis=(1, 2))
        ss = jnp.sum(xb * xb, axis=(1, 2))

        @pl.when(h == 0)
        def _():
            out_ref[:, 0, :] = s
            out_ref[:, 1, :] = ss

        @pl.when(h > 0)
        def _():
            out_ref[:, 0, :] += s
            out_ref[:, 1, :] += ss

    partial = pl.pallas_call(
        stats_body,
        grid=(n_h,),
        in_specs=[pl.BlockSpec((B, BH, W, C), lambda h: (0, h, 0, 0))],
        out_specs=pl.BlockSpec((B, 2, C), lambda h: (0, 0, 0)),
        out_shape=jax.ShapeDtypeStruct((B, 2, C), jnp.float32),
    )(x)

    def ar_body(p_ref, out_ref, comm_ref, send_sems, recv_sems):
        my = lax.axis_index("i")
        rdmas = []
        for k in range(1, N_DEV):
            tgt = lax.rem(my + k, N_DEV)
            rdma = pltpu.make_async_remote_copy(
                src_ref=p_ref,
                dst_ref=comm_ref.at[k - 1],
                send_sem=send_sems.at[k - 1],
                recv_sem=recv_sems.at[k - 1],
                device_id=(tgt,),
                device_id_type=pl.DeviceIdType.MESH,
            )
            rdma.start()
            rdmas.append(rdma)
        for rdma in rdmas:
            rdma.wait_recv()
        for rdma in rdmas:
            rdma.wait_send()

        tot = p_ref[...] + comm_ref[0] + comm_ref[1] + comm_ref[2]
        s = tot[:, 0, :]
        ss = tot[:, 1, :]
        mean = s * inv_n
        var = ss * inv_n - mean * mean
        rstd = lax.rsqrt(var + EPS)
        out_ref[:, 0, :] = mean
        out_ref[:, 1, :] = rstd

    stats = pl.pallas_call(
        ar_body,
        out_shape=jax.ShapeDtypeStruct((B, 2, C), jnp.float32),
        in_specs=[pl.BlockSpec(memory_space=pltpu.VMEM)],
        out_specs=pl.BlockSpec(memory_space=pltpu.VMEM),
        scratch_shapes=[
            pltpu.VMEM((N_DEV - 1, B, 2, C), jnp.float32),
            pltpu.SemaphoreType.DMA((N_DEV - 1,)),
            pltpu.SemaphoreType.DMA((N_DEV - 1,)),
        ],
        compiler_params=pltpu.CompilerParams(collective_id=0),
    )(partial)

    def apply_body(x_ref, st_ref, w_ref, out_ref):
        xb = x_ref[...]
        mean = st_ref[:, 0, :][:, None, None, :]
        rstd = st_ref[:, 1, :][:, None, None, :]
        hb = (xb - mean) * rstd
        a = hb * jax.nn.sigmoid(hb)
        a16 = a.astype(jnp.bfloat16).reshape(B * BH * W, C)
        w16 = w_ref[...].astype(jnp.bfloat16)
        o = jnp.dot(a16, w16, preferred_element_type=jnp.float32)
        out_ref[...] = o.reshape(B, BH, W, Cout).astype(jnp.bfloat16)

    return pl.pallas_call(
        apply_body,
        grid=(n_h,),
        in_specs=[
            pl.BlockSpec((B, BH, W, C), lambda h: (0, h, 0, 0)),
            pl.BlockSpec((B, 2, C), lambda h: (0, 0, 0)),
            pl.BlockSpec((C, Cout), lambda h: (0, 0)),
        ],
        out_specs=pl.BlockSpec((B, BH, W, Cout), lambda h: (0, h, 0, 0)),
        out_shape=jax.ShapeDtypeStruct((B, Hs, W, Cout), jnp.bfloat16),
    )(x, stats, Wp)


# baseline (device time: 77096 ns/iter reference)
import jax
import jax.numpy as jnp
from jax import lax
from jax.experimental import pallas as pl
from jax.experimental.pallas import tpu as pltpu

N_DEV = 4
EPS = 1e-5


def kernel(x, Wp):
    B, Hs, W, C = x.shape
    Cout = Wp.shape[1]
    BH = 32
    n_h = Hs // BH
    inv_n = 1.0 / float(N_DEV * Hs * W)

    def stats_body(x_ref, out_ref):
        h = pl.program_id(0)
        xb = x_ref[...]
        s = jnp.sum(xb, axis=(1, 2))
        ss = jnp.sum(xb * xb, axis=(1, 2))

        @pl.when(h == 0)
        def _():
            out_ref[:, 0, :] = s
            out_ref[:, 1, :] = ss

        @pl.when(h > 0)
        def _():
            out_ref[:, 0, :] += s
            out_ref[:, 1, :] += ss

    partial = pl.pallas_call(
        stats_body,
        grid=(n_h,),
        in_specs=[pl.BlockSpec((B, BH, W, C), lambda h: (0, h, 0, 0))],
        out_specs=pl.BlockSpec((B, 2, C), lambda h: (0, 0, 0)),
        out_shape=jax.ShapeDtypeStruct((B, 2, C), jnp.float32),
    )(x)

    def ar_body(p_ref, out_ref, comm_ref, send_sems, recv_sems):
        my = lax.axis_index("i")
        rdmas = []
        for k in range(1, N_DEV):
            tgt = lax.rem(my + k, N_DEV)
            rdma = pltpu.make_async_remote_copy(
                src_ref=p_ref,
                dst_ref=comm_ref.at[k - 1],
                send_sem=send_sems.at[k - 1],
                recv_sem=recv_sems.at[k - 1],
                device_id=(tgt,),
                device_id_type=pl.DeviceIdType.MESH,
            )
            rdma.start()
            rdmas.append(rdma)
        for rdma in rdmas:
            rdma.wait_recv()
        for rdma in rdmas:
            rdma.wait_send()

        tot = p_ref[...] + comm_ref[0] + comm_ref[1] + comm_ref[2]
        s = tot[:, 0, :]
        ss = tot[:, 1, :]
        mean = s * inv_n
        var = ss * inv_n - mean * mean
        rstd = lax.rsqrt(var + EPS)
        out_ref[:, 0, :] = mean
        out_ref[:, 1, :] = rstd

    stats = pl.pallas_call(
        ar_body,
        out_shape=jax.ShapeDtypeStruct((B, 2, C), jnp.float32),
        in_specs=[pl.BlockSpec(memory_space=pltpu.VMEM)],
        out_specs=pl.BlockSpec(memory_space=pltpu.VMEM),
        scratch_shapes=[
            pltpu.VMEM((N_DEV - 1, B, 2, C), jnp.float32),
            pltpu.SemaphoreType.DMA((N_DEV - 1,)),
            pltpu.SemaphoreType.DMA((N_DEV - 1,)),
        ],
    )(partial)

    BHA = 16
    n_ha = Hs // BHA

    def apply_body(x_ref, st_ref, w_ref, out_ref):
        xb = x_ref[...]
        mean = st_ref[:, 0, :][:, None, None, :]
        rstd = st_ref[:, 1, :][:, None, None, :]
        hb = (xb - mean) * rstd
        a = hb * jax.nn.sigmoid(hb)
        a16 = a.astype(jnp.bfloat16).reshape(B * BHA * W, C)
        w16 = w_ref[...].astype(jnp.bfloat16)
        o = jnp.dot(a16, w16, preferred_element_type=jnp.float32)
        out_ref[...] = o.reshape(B, BHA, W, Cout).astype(jnp.bfloat16)

    return pl.pallas_call(
        apply_body,
        grid=(n_ha,),
        in_specs=[
            pl.BlockSpec((B, BHA, W, C), lambda h: (0, h, 0, 0)),
            pl.BlockSpec((B, 2, C), lambda h: (0, 0, 0)),
            pl.BlockSpec((C, Cout), lambda h: (0, 0)),
        ],
        out_specs=pl.BlockSpec((B, BHA, W, Cout), lambda h: (0, h, 0, 0)),
        out_shape=jax.ShapeDtypeStruct((B, Hs, W, Cout), jnp.bfloat16),
    )(x, stats, Wp)
